# Initial kernel scaffold; baseline (speedup 1.0000x reference)
import jax
import jax.numpy as jnp
from jax.experimental import pallas as pl


def kernel(H_d, H_t, W1, W2):
    raise NotImplementedError("write your pallas kernel here")


# speedup vs baseline: 22.1562x; 22.1562x over previous
import functools

import jax
import jax.numpy as jnp
from jax.experimental import pallas as pl

D_NUM, T_NUM, D_DIM, T_DIM = 4096, 4096, 512, 512
UNITS = 256
K = 32
BLOCK_ROWS = 256
HI_BITS = 0x3F800001
SEARCH_ITERS = 30


def _proj_kernel(hd_ref, w1_ref, ht_ref, w2_ref, o1_ref, o2_ref):
    o1_ref[...] = jnp.dot(hd_ref[...], w1_ref[...])
    o2_ref[...] = jnp.dot(ht_ref[...], w2_ref[...])


def _row_topk_mask(R):
    B, T = R.shape
    bits = jax.lax.bitcast_convert_type(R, jnp.int32)

    def body(_, carry):
        lo, hi = carry
        mid = jax.lax.shift_right_logical(lo + hi, 1)
        cnt = jnp.sum(jnp.where(bits >= mid, 1.0, 0.0), axis=1, keepdims=True)
        ge = cnt >= K
        return jnp.where(ge, mid, lo), jnp.where(ge, hi, mid)

    lo0 = jnp.zeros((B, 1), jnp.int32)
    hi0 = jnp.full((B, 1), HI_BITS, jnp.int32)
    t, _ = jax.lax.fori_loop(0, SEARCH_ITERS, body, (lo0, hi0))

    gt = bits > t
    eq = bits == t
    n_gt = jnp.sum(jnp.where(gt, 1.0, 0.0), axis=1, keepdims=True)
    need = K - n_gt

    col = jax.lax.broadcasted_iota(jnp.int32, (B, T), 1)
    x = jnp.where(eq, 1.0, 0.0)
    s = 1
    while s < T:
        x = x + jnp.where(col >= s, jnp.roll(x, s, axis=1), 0.0)
        s *= 2
    prefix_excl = x - jnp.where(eq, 1.0, 0.0)

    return gt | (eq & (prefix_excl < need))


def _sim_kernel(hd_ref, ht_ref, r_ref, rf_ref):
    S = jax.lax.dot_general(hd_ref[...], ht_ref[...],
                            (((1,), (1,)), ((), ())))
    R = jax.nn.sigmoid(S)
    r_ref[...] = R
    mask = _row_topk_mask(R)
    rf_ref[...] = jnp.where(mask, R, 0.0)


@functools.partial(jax.jit)
def kernel(H_d, H_t, W1, W2):
    Hd, Ht = pl.pallas_call(
        _proj_kernel,
        out_shape=(jax.ShapeDtypeStruct((D_NUM, UNITS), jnp.float32),
                   jax.ShapeDtypeStruct((T_NUM, UNITS), jnp.float32)),
    )(H_d, W1, H_t, W2)

    grid = D_NUM // BLOCK_ROWS
    R, R_flt = pl.pallas_call(
        _sim_kernel,
        grid=(grid,),
        in_specs=[
            pl.BlockSpec((BLOCK_ROWS, UNITS), lambda i: (i, 0)),
            pl.BlockSpec((T_NUM, UNITS), lambda i: (0, 0)),
        ],
        out_specs=[
            pl.BlockSpec((BLOCK_ROWS, T_NUM), lambda i: (i, 0)),
            pl.BlockSpec((BLOCK_ROWS, T_NUM), lambda i: (i, 0)),
        ],
        out_shape=(jax.ShapeDtypeStruct((D_NUM, T_NUM), jnp.float32),
                   jax.ShapeDtypeStruct((D_NUM, T_NUM), jnp.float32)),
    )(Hd, Ht)

    z = jnp.zeros(())
    return (R, R_flt, z, z, z, z)


# speedup vs baseline: 47.9794x; 2.1655x over previous
import functools

import jax
import jax.numpy as jnp
from jax.experimental import pallas as pl

D_NUM, T_NUM, D_DIM, T_DIM = 4096, 4096, 512, 512
UNITS = 256
K = 32
BLOCK_ROWS = 256
HI_BITS = 0x3F800001
SEARCH_ITERS = 30


def _proj_kernel(hd_ref, w1_ref, ht_ref, w2_ref, o1_ref, o2_ref):
    o1_ref[...] = jnp.dot(hd_ref[...], w1_ref[...])
    o2_ref[...] = jnp.dot(ht_ref[...], w2_ref[...])


def _row_topk_mask(R):
    B, T = R.shape
    bits = jax.lax.bitcast_convert_type(R, jnp.int32)

    cmax = jnp.max(R.reshape(B, T // 128, 128), axis=2)
    lo0 = jax.lax.bitcast_convert_type(
        jnp.min(cmax, axis=1, keepdims=True), jnp.int32)
    hi0 = jax.lax.bitcast_convert_type(
        jnp.max(cmax, axis=1, keepdims=True), jnp.int32) + 1

    def cond(carry):
        lo, hi = carry
        return jnp.any(hi - lo > 1)

    def body(carry):
        lo, hi = carry
        mid = jax.lax.shift_right_logical(lo + hi, 1)
        cnt = jnp.sum(jnp.where(bits >= mid, 1.0, 0.0), axis=1, keepdims=True)
        ge = cnt >= K
        return jnp.where(ge, mid, lo), jnp.where(ge, hi, mid)

    t, _ = jax.lax.while_loop(cond, body, (lo0, hi0))

    gt = bits > t
    eq = bits == t
    n_gt = jnp.sum(jnp.where(gt, 1.0, 0.0), axis=1, keepdims=True)
    need = K - n_gt

    col = jax.lax.broadcasted_iota(jnp.int32, (B, T), 1)
    x = jnp.where(eq, 1.0, 0.0)
    s = 1
    while s < T:
        x = x + jnp.where(col >= s, jnp.roll(x, s, axis=1), 0.0)
        s *= 2
    prefix_excl = x - jnp.where(eq, 1.0, 0.0)

    return gt | (eq & (prefix_excl < need))


def _sim_kernel(hd_ref, ht_ref, r_ref, rf_ref):
    S = jax.lax.dot_general(hd_ref[...], ht_ref[...],
                            (((1,), (1,)), ((), ())))
    R = jax.nn.sigmoid(S)
    r_ref[...] = R
    mask = _row_topk_mask(R)
    rf_ref[...] = jnp.where(mask, R, 0.0)


@functools.partial(jax.jit)
def kernel(H_d, H_t, W1, W2):
    Hd, Ht = pl.pallas_call(
        _proj_kernel,
        out_shape=(jax.ShapeDtypeStruct((D_NUM, UNITS), jnp.float32),
                   jax.ShapeDtypeStruct((T_NUM, UNITS), jnp.float32)),
    )(H_d, W1, H_t, W2)

    grid = D_NUM // BLOCK_ROWS
    R, R_flt = pl.pallas_call(
        _sim_kernel,
        grid=(grid,),
        in_specs=[
            pl.BlockSpec((BLOCK_ROWS, UNITS), lambda i: (i, 0)),
            pl.BlockSpec((T_NUM, UNITS), lambda i: (0, 0)),
        ],
        out_specs=[
            pl.BlockSpec((BLOCK_ROWS, T_NUM), lambda i: (i, 0)),
            pl.BlockSpec((BLOCK_ROWS, T_NUM), lambda i: (i, 0)),
        ],
        out_shape=(jax.ShapeDtypeStruct((D_NUM, T_NUM), jnp.float32),
                   jax.ShapeDtypeStruct((D_NUM, T_NUM), jnp.float32)),
    )(Hd, Ht)

    z = jnp.zeros(())
    return (R, R_flt, z, z, z, z)


# speedup vs baseline: 90.8932x; 1.8944x over previous
import functools

import jax
import jax.numpy as jnp
from jax.experimental import pallas as pl

D_NUM, T_NUM, D_DIM, T_DIM = 4096, 4096, 512, 512
UNITS = 256
K = 32
BLOCK_ROWS = 256
HI_BITS = 0x3F800001
SEARCH_ITERS = 30


def _proj_kernel(hd_ref, w1_ref, ht_ref, w2_ref, o1_ref, o2_ref):
    o1_ref[...] = jnp.dot(hd_ref[...], w1_ref[...])
    o2_ref[...] = jnp.dot(ht_ref[...], w2_ref[...])


def _row_topk_mask(R):
    B, T = R.shape
    bits = jax.lax.bitcast_convert_type(R, jnp.int32)

    cmax = jnp.max(R.reshape(B, T // 128, 128), axis=2)
    lo0 = jax.lax.bitcast_convert_type(
        jnp.min(cmax, axis=1, keepdims=True), jnp.int32)
    hi0 = jax.lax.bitcast_convert_type(
        jnp.max(cmax, axis=1, keepdims=True), jnp.int32) + 1

    def cond(carry):
        lo, hi = carry
        return jnp.any(hi - lo > 1)

    def body(carry):
        lo, hi = carry
        mid = jax.lax.shift_right_logical(lo + hi, 1)
        cnt = jnp.sum(jnp.where(bits >= mid, 1.0, 0.0), axis=1, keepdims=True)
        ge = cnt >= K
        return jnp.where(ge, mid, lo), jnp.where(ge, hi, mid)

    t, _ = jax.lax.while_loop(cond, body, (lo0, hi0))

    gt = bits > t
    eq = bits == t
    n_gt = jnp.sum(jnp.where(gt, 1.0, 0.0), axis=1, keepdims=True)
    need = K - n_gt

    eqb = jnp.where(eq, 1.0, 0.0).astype(jnp.bfloat16)

    SW = 256
    ir = jax.lax.broadcasted_iota(jnp.int32, (SW, SW), 0)
    ic = jax.lax.broadcasted_iota(jnp.int32, (SW, SW), 1)
    L2 = jnp.where((ir <= ic) & ((ir >> 7) == (ic >> 7)), 1.0, 0.0
                   ).astype(jnp.bfloat16)

    ps = []
    for c in range(T // SW):
        sl = jax.lax.slice_in_dim(eqb, c * SW, (c + 1) * SW, axis=1)
        ps.append(jax.lax.dot_general(
            sl, L2, (((1,), (0,)), ((), ())),
            preferred_element_type=jnp.float32))
    P = jnp.concatenate(ps, axis=1)

    NC = T // 128
    ptot = jnp.concatenate(
        [jax.lax.slice_in_dim(P, c * 128 + 127, c * 128 + 128, axis=1)
         for c in range(NC)], axis=1)

    ia = jax.lax.broadcasted_iota(jnp.int32, (NC, NC), 0)
    ib = jax.lax.broadcasted_iota(jnp.int32, (NC, NC), 1)
    SL = jnp.where(ia < ib, 1.0, 0.0).astype(jnp.bfloat16)
    coarse = jax.lax.dot_general(
        ptot.astype(jnp.bfloat16), SL, (((1,), (0,)), ((), ())),
        preferred_element_type=jnp.float32)
    coarse = jnp.minimum(coarse, 64.0)
    offs = jnp.concatenate(
        [jnp.broadcast_to(jax.lax.slice_in_dim(coarse, c, c + 1, axis=1),
                          (B, 128)) for c in range(NC)], axis=1)

    return gt | (eq & ((P + offs) <= need))


def _sim_kernel(hd_ref, ht_ref, r_ref, rf_ref):
    S = jax.lax.dot_general(hd_ref[...], ht_ref[...],
                            (((1,), (1,)), ((), ())))
    R = jax.nn.sigmoid(S)
    r_ref[...] = R
    mask = _row_topk_mask(R)
    rf_ref[...] = jnp.where(mask, R, 0.0)


@functools.partial(jax.jit)
def kernel(H_d, H_t, W1, W2):
    Hd, Ht = pl.pallas_call(
        _proj_kernel,
        out_shape=(jax.ShapeDtypeStruct((D_NUM, UNITS), jnp.float32),
                   jax.ShapeDtypeStruct((T_NUM, UNITS), jnp.float32)),
    )(H_d, W1, H_t, W2)

    grid = D_NUM // BLOCK_ROWS
    R, R_flt = pl.pallas_call(
        _sim_kernel,
        grid=(grid,),
        in_specs=[
            pl.BlockSpec((BLOCK_ROWS, UNITS), lambda i: (i, 0)),
            pl.BlockSpec((T_NUM, UNITS), lambda i: (0, 0)),
        ],
        out_specs=[
            pl.BlockSpec((BLOCK_ROWS, T_NUM), lambda i: (i, 0)),
            pl.BlockSpec((BLOCK_ROWS, T_NUM), lambda i: (i, 0)),
        ],
        out_shape=(jax.ShapeDtypeStruct((D_NUM, T_NUM), jnp.float32),
                   jax.ShapeDtypeStruct((D_NUM, T_NUM), jnp.float32)),
    )(Hd, Ht)

    z = jnp.zeros(())
    return (R, R_flt, z, z, z, z)
